# baseline (speedup 1.0000x reference)
import functools

import jax
import jax.numpy as jnp
from jax import lax
from jax.experimental import pallas as pl
from jax.experimental.pallas import tpu as pltpu

_BOUNDS = 1.6
_C = 16
_M = 2000


def _pack_plane(g):
    gt = jnp.transpose(g, (1, 2, 0))
    h, w, c = gt.shape
    gtp = jnp.concatenate([gt, gt[:, -1:, :], gt[:, -1:, :]], axis=1)
    a = gtp[:, 0:w, :].reshape(h, w // 2, 2 * c)
    b = gtp[:, 2:w + 2, :].reshape(h, w // 2, 2 * c)
    quad = jnp.concatenate([a, b], axis=-1)
    qnext = jnp.concatenate([quad[1:], quad[-1:]], axis=0)
    p = jnp.concatenate([quad, qnext], axis=-1)
    return p.reshape(h * (w // 2), 1, 8 * _C)


def _cell(coord, n):
    x = jnp.clip((coord + 1.0) * 0.5 * (n - 1), 0.0, float(n - 1))
    x0 = jnp.floor(x)
    return x0.astype(jnp.int32), x - x0


def _plane_stream(xc, yc, w, h):
    x0, wx = _cell(xc, w)
    y0, wy = _cell(yc, h)
    f = y0 * jnp.int32(w // 2) + (x0 >> 1)
    par = (x0 & 1) == 1
    zero = jnp.zeros_like(wx)
    one = jnp.float32(1.0)
    xw = [jnp.where(par, zero, one - wx),
          jnp.where(par, one - wx, wx),
          jnp.where(par, wx, zero),
          zero]
    nb = wx.shape[0] // _M
    w8t = jnp.stack([(q * (one - wy)).reshape(nb, _M) for q in xw]
                    + [(q * wy).reshape(nb, _M) for q in xw], axis=1)
    return f, w8t


def _gather_kernel(si, f0, f1, f2, w0, w1, w2, p0, p1, p2, out,
                   t0, t1, t2):
    for f_ref, p_ref, tile in ((f0, p0, t0), (f1, p1, t1), (f2, p2, t2)):
        for mi in range(_M):
            tile[mi] = p_ref[f_ref[0, 0, mi], 0]
    sel = (lax.broadcasted_iota(jnp.int32, (8, 8 * _C), 1) // _C
           == lax.broadcasted_iota(jnp.int32, (8, 8 * _C), 0)
           ).astype(jnp.float32)
    fold = (lax.broadcasted_iota(jnp.int32, (8 * _C, 3 * _C), 0) % _C
            == lax.broadcasted_iota(jnp.int32, (8 * _C, 3 * _C), 1) - si * _C
            ).astype(jnp.float32)
    acc = None
    for w_ref, tile in ((w0, t0), (w1, t1), (w2, t2)):
        wexp = lax.dot_general(w_ref[0], sel, (((0,), (0,)), ((), ())),
                               preferred_element_type=jnp.float32)
        term = tile[...] * wexp
        acc = term if acc is None else acc + term
    out[...] = jnp.dot(acc, fold, preferred_element_type=jnp.float32)


def _scale_call(packs, fs, w8s, n, si):
    nb = n // _M
    grid = (nb,)
    f3d = [f.reshape(nb, 1, _M) for f in fs]

    def fmap(i):
        return (i, 0, 0)

    def bmap(i):
        return (i, 0)

    return pl.pallas_call(
        functools.partial(_gather_kernel, si),
        grid=grid,
        in_specs=(
            [pl.BlockSpec((1, 1, _M), fmap, memory_space=pltpu.SMEM)] * 3
            + [pl.BlockSpec((1, 8, _M), fmap)] * 3
            + [pl.BlockSpec(memory_space=pltpu.VMEM)] * 3
        ),
        out_specs=pl.BlockSpec((_M, 3 * _C), bmap),
        out_shape=jax.ShapeDtypeStruct((n, 3 * _C), jnp.float32),
        scratch_shapes=[pltpu.VMEM((_M, 8 * _C), jnp.float32)] * 3,
        compiler_params=pltpu.CompilerParams(
            dimension_semantics=("arbitrary",),
            vmem_limit_bytes=64 * 1024 * 1024,
        ),
    )(*f3d, *w8s, *packs)


def kernel(pts, pts_time, timestamps, duration,
           sp_s0_c0, sp_s0_c1, sp_s0_c2, tp_s0_c0, tp_s0_c1, tp_s0_c2,
           sp_s1_c0, sp_s1_c1, sp_s1_c2, tp_s1_c0, tp_s1_c1, tp_s1_c2,
           sp_s2_c0, sp_s2_c1, sp_s2_c2, tp_s2_c0, tp_s2_c1, tp_s2_c2):
    space_grids = [[sp_s0_c0, sp_s0_c1, sp_s0_c2],
                   [sp_s1_c0, sp_s1_c1, sp_s1_c2],
                   [sp_s2_c0, sp_s2_c1, sp_s2_c2]]
    time_grids = [[tp_s0_c0, tp_s0_c1, tp_s0_c2],
                  [tp_s1_c0, tp_s1_c1, tp_s1_c2],
                  [tp_s2_c0, tp_s2_c1, tp_s2_c2]]
    n = pts.shape[0]
    aabb0 = jnp.full((3,), _BOUNDS, jnp.float32)
    aabb1 = -aabb0
    pts_n = (pts - aabb0) * (2.0 / (aabb1 - aabb0)) - 1.0
    ptt_n = (pts_time - aabb0) * (2.0 / (aabb1 - aabb0)) - 1.0
    d = jnp.float32(duration)
    t_n = (2.0 * timestamps * d / (d - 1.0) - 1.0)[:, 0]

    space_combs = [(0, 1), (0, 2), (1, 2)]
    space_out, time_out = [], []
    for si in range(3):
        packs, fs, w8s = [], [], []
        for ci, (a, b) in enumerate(space_combs):
            g = space_grids[si][ci]
            _, hh, ww = g.shape
            f, w8 = _plane_stream(pts_n[:, a], pts_n[:, b], ww, hh)
            packs.append(_pack_plane(g))
            fs.append(f)
            w8s.append(w8)
        space_out.append(_scale_call(packs, fs, w8s, n, si))

        packs, fs, w8s = [], [], []
        for ci in range(3):
            g = time_grids[si][ci]
            _, hh, ww = g.shape
            f, w8 = _plane_stream(ptt_n[:, ci], t_n, ww, hh)
            packs.append(_pack_plane(g))
            fs.append(f)
            w8s.append(w8)
        time_out.append(_scale_call(packs, fs, w8s, n, si))

    space_features = space_out[0] + space_out[1] + space_out[2]
    time_features = time_out[0] + time_out[1] + time_out[2]
    return (space_features, time_features)


# speedup vs baseline: 1.0462x; 1.0462x over previous
import functools

import jax
import jax.numpy as jnp
from jax import lax
from jax.experimental import pallas as pl
from jax.experimental.pallas import tpu as pltpu

_BOUNDS = 1.6
_C = 16
_M = 2000


def _pack_plane(g):
    gt = jnp.transpose(g, (1, 2, 0))
    h, w, c = gt.shape
    gtp = jnp.concatenate([gt, gt[:, -1:, :], gt[:, -1:, :]], axis=1)
    a = gtp[:, 0:w, :].reshape(h, w // 2, 2 * c)
    b = gtp[:, 2:w + 2, :].reshape(h, w // 2, 2 * c)
    quad = jnp.concatenate([a, b], axis=-1)
    qnext = jnp.concatenate([quad[1:], quad[-1:]], axis=0)
    p = jnp.concatenate([quad, qnext], axis=-1)
    return p.reshape(h * (w // 2), 1, 8 * _C)


def _cell(coord, n):
    x = jnp.clip((coord + 1.0) * 0.5 * (n - 1), 0.0, float(n - 1))
    x0 = jnp.floor(x)
    return x0.astype(jnp.int32), x - x0


def _plane_stream(xc, yc, w, h):
    x0, wx = _cell(xc, w)
    y0, wy = _cell(yc, h)
    f = y0 * jnp.int32(w // 2) + (x0 >> 1)
    par = (x0 & 1) == 1
    zero = jnp.zeros_like(wx)
    one = jnp.float32(1.0)
    xw = [jnp.where(par, zero, one - wx),
          jnp.where(par, one - wx, wx),
          jnp.where(par, wx, zero),
          zero]
    nb = wx.shape[0] // _M
    w8t = jnp.stack([(q * (one - wy)).reshape(nb, _M) for q in xw]
                    + [(q * wy).reshape(nb, _M) for q in xw], axis=1)
    return f, w8t


def _gather_kernel(si, f0, f1, f2, w0, w1, w2, p0, p1, p2, *rest):
    if si == 0:
        (out, t0, t1, t2), prev = rest, None
    else:
        (prev, out, t0, t1, t2) = rest
    for f_ref, p_ref, tile in ((f0, p0, t0), (f1, p1, t1), (f2, p2, t2)):
        for mi in range(_M):
            tile[mi] = p_ref[f_ref[0, 0, mi], 0]
    sel = (lax.broadcasted_iota(jnp.int32, (8, 8 * _C), 1) // _C
           == lax.broadcasted_iota(jnp.int32, (8, 8 * _C), 0)
           ).astype(jnp.float32)
    fold = (lax.broadcasted_iota(jnp.int32, (8 * _C, 3 * _C), 0) % _C
            == lax.broadcasted_iota(jnp.int32, (8 * _C, 3 * _C), 1) - si * _C
            ).astype(jnp.float32)
    acc = None
    for w_ref, tile in ((w0, t0), (w1, t1), (w2, t2)):
        wexp = lax.dot_general(w_ref[0], sel, (((0,), (0,)), ((), ())),
                               preferred_element_type=jnp.float32)
        term = tile[...] * wexp
        acc = term if acc is None else acc + term
    res = jnp.dot(acc, fold, preferred_element_type=jnp.float32)
    out[...] = res if prev is None else prev[...] + res


def _scale_call(packs, fs, w8s, n, si, prev):
    nb = n // _M
    grid = (nb,)
    f3d = [f.reshape(nb, 1, _M) for f in fs]

    def fmap(i):
        return (i, 0, 0)

    def bmap(i):
        return (i, 0)

    specs = ([pl.BlockSpec((1, 1, _M), fmap, memory_space=pltpu.SMEM)] * 3
             + [pl.BlockSpec((1, 8, _M), fmap)] * 3
             + [pl.BlockSpec(memory_space=pltpu.VMEM)] * 3)
    args = [*f3d, *w8s, *packs]
    aliases = {}
    if prev is not None:
        specs.append(pl.BlockSpec((_M, 3 * _C), bmap))
        args.append(prev)
        aliases = {9: 0}
    return pl.pallas_call(
        functools.partial(_gather_kernel, si),
        grid=grid,
        in_specs=specs,
        out_specs=pl.BlockSpec((_M, 3 * _C), bmap),
        out_shape=jax.ShapeDtypeStruct((n, 3 * _C), jnp.float32),
        input_output_aliases=aliases,
        scratch_shapes=[pltpu.VMEM((_M, 8 * _C), jnp.float32)] * 3,
        compiler_params=pltpu.CompilerParams(
            dimension_semantics=("arbitrary",),
            vmem_limit_bytes=64 * 1024 * 1024,
        ),
    )(*args)


def kernel(pts, pts_time, timestamps, duration,
           sp_s0_c0, sp_s0_c1, sp_s0_c2, tp_s0_c0, tp_s0_c1, tp_s0_c2,
           sp_s1_c0, sp_s1_c1, sp_s1_c2, tp_s1_c0, tp_s1_c1, tp_s1_c2,
           sp_s2_c0, sp_s2_c1, sp_s2_c2, tp_s2_c0, tp_s2_c1, tp_s2_c2):
    space_grids = [[sp_s0_c0, sp_s0_c1, sp_s0_c2],
                   [sp_s1_c0, sp_s1_c1, sp_s1_c2],
                   [sp_s2_c0, sp_s2_c1, sp_s2_c2]]
    time_grids = [[tp_s0_c0, tp_s0_c1, tp_s0_c2],
                  [tp_s1_c0, tp_s1_c1, tp_s1_c2],
                  [tp_s2_c0, tp_s2_c1, tp_s2_c2]]
    n = pts.shape[0]
    aabb0 = jnp.full((3,), _BOUNDS, jnp.float32)
    aabb1 = -aabb0
    pts_n = (pts - aabb0) * (2.0 / (aabb1 - aabb0)) - 1.0
    ptt_n = (pts_time - aabb0) * (2.0 / (aabb1 - aabb0)) - 1.0
    d = jnp.float32(duration)
    t_n = (2.0 * timestamps * d / (d - 1.0) - 1.0)[:, 0]

    space_combs = [(0, 1), (0, 2), (1, 2)]
    space_features = None
    time_features = None
    for si in range(3):
        packs, fs, w8s = [], [], []
        for ci, (a, b) in enumerate(space_combs):
            g = space_grids[si][ci]
            _, hh, ww = g.shape
            f, w8 = _plane_stream(pts_n[:, a], pts_n[:, b], ww, hh)
            packs.append(_pack_plane(g))
            fs.append(f)
            w8s.append(w8)
        space_features = _scale_call(packs, fs, w8s, n, si, space_features)

        packs, fs, w8s = [], [], []
        for ci in range(3):
            g = time_grids[si][ci]
            _, hh, ww = g.shape
            f, w8 = _plane_stream(ptt_n[:, ci], t_n, ww, hh)
            packs.append(_pack_plane(g))
            fs.append(f)
            w8s.append(w8)
        time_features = _scale_call(packs, fs, w8s, n, si, time_features)

    return (space_features, time_features)
